# Initial kernel scaffold; baseline (speedup 1.0000x reference)
import jax
import jax.numpy as jnp
from jax.experimental import pallas as pl


def kernel(x, edge_idx, W0, b0, W1, b1, W2, b2, Wl, bl):
    raise NotImplementedError("write your pallas kernel here")


# speedup vs baseline: 8.8084x; 8.8084x over previous
import functools

import jax
import jax.numpy as jnp
from jax import lax
from jax.experimental import pallas as pl
from jax.experimental.pallas import tpu as pltpu
from jax.experimental.pallas import tpu_sc as plsc

_N, _D, _H, _C, _E = 10000, 128, 128, 40, 320000

_K = 128
_NC, _NS = 2, 16
_NW = _NC * _NS
_NBLK_E = -(-_E // (_NW * _K))
_NBLK_E += _NBLK_E % 2
_PER_W = _NBLK_E * _K
_E_PAD = _PER_W * _NW
_NP = (-(-(_N + 1) // _K)) * _K
_NBLK_N = _NP // _K
_RT = 640
_RP = _RT * _NS
_L = 16


@functools.cache
def _mesh():
    return plsc.VectorSubcoreMesh(
        core_axis_name="c", subcore_axis_name="s",
        num_cores=_NC, num_subcores=_NS)


def _fill_2d(ref, ncols, val):
    def body(i, carry):
        for j in range(ncols // _L):
            ref[i, pl.ds(j * _L, _L)] = jnp.full((_L,), val, jnp.float32)
        return carry
    lax.fori_loop(0, _K, body, 0)


def _fill_1d(ref, n, val):
    def body(i, carry):
        ref[pl.ds(i * _L, _L)] = jnp.full((_L,), val, jnp.float32)
        return carry
    lax.fori_loop(0, n // _L, body, 0)


def _deg_call(dst_p):
    @functools.partial(
        pl.kernel,
        out_type=jax.ShapeDtypeStruct((_NW, _NP), jnp.float32),
        mesh=_mesh(),
        compiler_params=pltpu.CompilerParams(needs_layout_passes=False),
        scratch_types=[
            pltpu.VMEM((_K,), jnp.int32),
            pltpu.VMEM((_NP,), jnp.float32),
        ],
    )
    def deg_k(dst_hbm, out_hbm, didx, degv):
        c = lax.axis_index("c")
        s = lax.axis_index("s")
        w = c * _NS + s
        _fill_1d(degv, _NP, 0.0)
        base = w * _PER_W
        ones = jnp.ones((_L,), jnp.float32)

        def body(b, carry):
            off = pl.multiple_of(base + b * _K, _K)
            pltpu.sync_copy(dst_hbm.at[pl.ds(off, _K)], didx)
            for i in range(_K // _L):
                idx = didx[pl.ds(i * _L, _L)]
                plsc.addupdate_scatter(degv, [idx], ones)
            return carry

        lax.fori_loop(0, _NBLK_E, body, 0)
        pltpu.sync_copy(degv, out_hbm.at[w])

    return deg_k(dst_p)


def _prop_scalar_call(q0, dis1, src_p, dst_p):
    @functools.partial(
        pl.kernel,
        out_type=(jax.ShapeDtypeStruct((_NC, _RP, _D), jnp.float32),
                  jax.ShapeDtypeStruct((_NW, _NP), jnp.float32)),
        mesh=_mesh(),
        compiler_params=pltpu.CompilerParams(needs_layout_passes=False),
        scratch_types=[
            pltpu.VMEM((_K,), jnp.int32),
            pltpu.VMEM((_K,), jnp.int32),
            pltpu.VMEM((_K, _D), jnp.float32),
            pltpu.VMEM((_NP,), jnp.float32),
            pltpu.VMEM((_NP,), jnp.float32),
            pltpu.VMEM_SHARED((_RP, _D), jnp.float32),
            pltpu.SemaphoreType.DMA,
        ],
    )
    def prop_k(q_hbm, dis_hbm, src_hbm, dst_hbm, agg_hbm, s_hbm,
               sidx, didx, rows, disv, sv, acc, sem):
        c = lax.axis_index("c")
        s = lax.axis_index("s")
        w = c * _NS + s
        _fill_2d(rows, _D, 0.0)
        for j in range(_RT // _K):
            pltpu.sync_copy(rows, acc.at[pl.ds(s * _RT + j * _K, _K)])
        _fill_1d(sv, _NP, 0.0)
        pltpu.sync_copy(dis_hbm, disv)
        plsc.subcore_barrier()
        base = w * _PER_W

        def body(b, carry):
            off = pl.multiple_of(base + b * _K, _K)
            pltpu.sync_copy(src_hbm.at[pl.ds(off, _K)], sidx)
            pltpu.sync_copy(dst_hbm.at[pl.ds(off, _K)], didx)
            g = pltpu.async_copy(q_hbm.at[sidx], rows, sem)
            for i in range(_K // _L):
                di = didx[pl.ds(i * _L, _L)]
                si = sidx[pl.ds(i * _L, _L)]
                vals = plsc.load_gather(disv, [di])
                plsc.addupdate_scatter(sv, [si], vals)
            g.wait()
            pltpu.sync_copy(rows, acc.at[didx], add=True)
            return carry

        lax.fori_loop(0, _NBLK_E, body, 0)
        plsc.subcore_barrier()
        pltpu.sync_copy(acc.at[pl.ds(s * _RT, _RT)],
                        agg_hbm.at[c, pl.ds(s * _RT, _RT)])
        pltpu.sync_copy(sv, s_hbm.at[w])

    return prop_k(q0, dis1, src_p, dst_p)


def _prop_call(q1, src_p, dst_p):
    @functools.partial(
        pl.kernel,
        out_type=jax.ShapeDtypeStruct((_NC, _RP, _D), jnp.float32),
        mesh=_mesh(),
        compiler_params=pltpu.CompilerParams(needs_layout_passes=False),
        scratch_types=[
            pltpu.VMEM((_K,), jnp.int32),
            pltpu.VMEM((_K,), jnp.int32),
            pltpu.VMEM((_K, _D), jnp.float32),
            pltpu.VMEM_SHARED((_RP, _D), jnp.float32),
            pltpu.SemaphoreType.DMA,
        ],
    )
    def prop_k(q_hbm, src_hbm, dst_hbm, agg_hbm, sidx, didx, rows, acc, sem):
        c = lax.axis_index("c")
        s = lax.axis_index("s")
        w = c * _NS + s
        _fill_2d(rows, _D, 0.0)
        for j in range(_RT // _K):
            pltpu.sync_copy(rows, acc.at[pl.ds(s * _RT + j * _K, _K)])
        plsc.subcore_barrier()
        base = w * _PER_W

        def body(b, carry):
            off = pl.multiple_of(base + b * _K, _K)
            pltpu.sync_copy(src_hbm.at[pl.ds(off, _K)], sidx)
            pltpu.sync_copy(dst_hbm.at[pl.ds(off, _K)], didx)
            pltpu.async_copy(q_hbm.at[sidx], rows, sem).wait()
            pltpu.sync_copy(rows, acc.at[didx], add=True)
            return carry

        lax.fori_loop(0, _NBLK_E, body, 0)
        plsc.subcore_barrier()
        pltpu.sync_copy(acc.at[pl.ds(s * _RT, _RT)],
                        agg_hbm.at[c, pl.ds(s * _RT, _RT)])

    return prop_k(q1, src_p, dst_p)


def _tc1_call(degp, x):
    def body(degp_ref, x_ref, q0_ref, dis16_ref):
        b = pl.program_id(0)
        deg = jnp.sum(degp_ref[...], axis=0)[:, None] + 1.0
        r16 = lax.broadcasted_iota(jnp.int32, (_K, 16), 0) + b * _K
        dis16 = jnp.where(r16 < _N,
                          lax.rsqrt(jnp.broadcast_to(deg, (_K, 16))), 0.0)
        dis16_ref[...] = dis16
        rD = lax.broadcasted_iota(jnp.int32, (_K, _D), 0) + b * _K
        q0_ref[...] = jnp.where(rD < _N, dis16[:, :1] * x_ref[...], 0.0)

    return pl.pallas_call(
        body,
        grid=(_NBLK_N,),
        in_specs=[pl.BlockSpec((_NW, _K), lambda b: (0, b)),
                  pl.BlockSpec((_K, _D), lambda b: (b, 0))],
        out_specs=[pl.BlockSpec((_K, _D), lambda b: (b, 0)),
                   pl.BlockSpec((_K, 16), lambda b: (b, 0))],
        out_shape=[jax.ShapeDtypeStruct((_NP, _D), jnp.float32),
                   jax.ShapeDtypeStruct((_NP, 16), jnp.float32)],
    )(degp, x)


def _tc2_call(aggp, q0, dis16, sp, W0, b0):
    def body(aggp_ref, q0_ref, dis16_ref, sp_ref, W0_ref, b0_ref,
             q1_ref, c16_ref):
        dis = dis16_ref[...]
        dcol = dis[:, :1]
        t = dcol * (aggp_ref[0] + aggp_ref[1] + q0_ref[...])
        h1 = jnp.maximum(
            jnp.dot(t, W0_ref[...], preferred_element_type=jnp.float32)
            + b0_ref[...], 0.0)
        q1_ref[...] = dcol * h1
        sblk = jnp.sum(sp_ref[...], axis=0)[:, None]
        c16_ref[...] = dis * (jnp.broadcast_to(sblk, (_K, 16)) + dis)

    return pl.pallas_call(
        body,
        grid=(_NBLK_N,),
        in_specs=[pl.BlockSpec((_NC, _K, _D), lambda b: (0, b, 0)),
                  pl.BlockSpec((_K, _D), lambda b: (b, 0)),
                  pl.BlockSpec((_K, 16), lambda b: (b, 0)),
                  pl.BlockSpec((_NW, _K), lambda b: (0, b)),
                  pl.BlockSpec((_D, _H), lambda b: (0, 0)),
                  pl.BlockSpec((1, _H), lambda b: (0, 0))],
        out_specs=[pl.BlockSpec((_K, _H), lambda b: (b, 0)),
                   pl.BlockSpec((_K, 16), lambda b: (b, 0))],
        out_shape=[jax.ShapeDtypeStruct((_NP, _H), jnp.float32),
                   jax.ShapeDtypeStruct((_NP, 16), jnp.float32)],
    )(aggp, q0, dis16, sp, W0, b0)


def _tc3_call(aggp, q1, dis16, c16, W1, b1, W2, b2, Wl, bl):
    def body(aggp_ref, q1_ref, dis16_ref, c16_ref, W1_ref, b1_ref,
             W2_ref, b2_ref, Wl_ref, bl_ref, out_ref, u_ref):
        b = pl.program_id(0)
        dcol = dis16_ref[:, :1]
        t = dcol * (aggp_ref[0] + aggp_ref[1] + q1_ref[...])
        h2 = jnp.maximum(
            jnp.dot(t, W1_ref[...], preferred_element_type=jnp.float32)
            + b1_ref[...], 0.0)
        contrib = jnp.sum(c16_ref[:, :1] * h2, axis=0, keepdims=True)

        @pl.when(b == 0)
        def _():
            u_ref[...] = jnp.zeros((1, _H), jnp.float32)

        u_ref[...] = u_ref[...] + contrib

        @pl.when(b == _NBLK_N - 1)
        def _():
            pooled = jnp.dot(u_ref[...], W2_ref[...],
                             preferred_element_type=jnp.float32) \
                + jnp.float32(_N) * b2_ref[...]
            out_ref[...] = jnp.dot(pooled, Wl_ref[...],
                                   preferred_element_type=jnp.float32) \
                + bl_ref[...]

    return pl.pallas_call(
        body,
        grid=(_NBLK_N,),
        in_specs=[pl.BlockSpec((_NC, _K, _D), lambda b: (0, b, 0)),
                  pl.BlockSpec((_K, _D), lambda b: (b, 0)),
                  pl.BlockSpec((_K, 16), lambda b: (b, 0)),
                  pl.BlockSpec((_K, 16), lambda b: (b, 0)),
                  pl.BlockSpec((_H, _H), lambda b: (0, 0)),
                  pl.BlockSpec((1, _H), lambda b: (0, 0)),
                  pl.BlockSpec((_H, _H), lambda b: (0, 0)),
                  pl.BlockSpec((1, _H), lambda b: (0, 0)),
                  pl.BlockSpec((_H, _C), lambda b: (0, 0)),
                  pl.BlockSpec((1, _C), lambda b: (0, 0))],
        out_specs=pl.BlockSpec((1, _C), lambda b: (0, 0)),
        out_shape=jax.ShapeDtypeStruct((1, _C), jnp.float32),
        scratch_shapes=[pltpu.VMEM((1, _H), jnp.float32)],
    )(aggp, q1, dis16, c16, W1, b1, W2, b2, Wl, bl)


def kernel(x, edge_idx, W0, b0, W1, b1, W2, b2, Wl, bl):
    src = edge_idx[0].astype(jnp.int32)
    dst = edge_idx[1].astype(jnp.int32)
    fill = jnp.full((_E_PAD - _E,), _N, jnp.int32)
    src_p = jnp.concatenate([src, fill])
    dst_p = jnp.concatenate([dst, fill])

    degp = _deg_call(dst_p)
    q0, dis16 = _tc1_call(degp, x)
    dis1 = dis16[:, 0]
    aggp, sp = _prop_scalar_call(q0, dis1, src_p, dst_p)
    q1, c16 = _tc2_call(aggp, q0, dis16, sp, W0, b0.reshape(1, _H))
    agg1p = _prop_call(q1, src_p, dst_p)
    return _tc3_call(agg1p, q1, dis16, c16, W1, b1.reshape(1, _H),
                     W2, b2.reshape(1, _H), Wl, bl.reshape(1, _C))


# speedup vs baseline: 11.3172x; 1.2848x over previous
import functools

import jax
import jax.numpy as jnp
from jax import lax
from jax.experimental import pallas as pl
from jax.experimental.pallas import tpu as pltpu
from jax.experimental.pallas import tpu_sc as plsc

_N, _D, _H, _C, _E = 10000, 128, 128, 40, 320000

_K = 128
_KE = 96
_NC, _NS = 2, 16
_NW = _NC * _NS
_NBLK_E = -(-_E // (_NW * _KE))
_NBLK_E += _NBLK_E % 2
_PER_W = _NBLK_E * _KE
_E_PAD = _PER_W * _NW
_NP = (-(-(_N + 1) // _K)) * _K
_NBLK_N = _NP // _K
_RT = 640
_RP = _RT * _NS
_L = 16


@functools.cache
def _mesh():
    return plsc.VectorSubcoreMesh(
        core_axis_name="c", subcore_axis_name="s",
        num_cores=_NC, num_subcores=_NS)


def _fill_2d(ref, nrows, ncols, val):
    def body(i, carry):
        for j in range(ncols // _L):
            ref[i, pl.ds(j * _L, _L)] = jnp.full((_L,), val, jnp.float32)
        return carry
    lax.fori_loop(0, nrows, body, 0)


def _fill_1d(ref, n, val):
    def body(i, carry):
        ref[pl.ds(i * _L, _L)] = jnp.full((_L,), val, jnp.float32)
        return carry
    lax.fori_loop(0, n // _L, body, 0)


def _deg_call(ei):
    @functools.partial(
        pl.kernel,
        out_type=jax.ShapeDtypeStruct((_NW, _NP), jnp.float32),
        mesh=_mesh(),
        compiler_params=pltpu.CompilerParams(needs_layout_passes=False),
        scratch_types=[
            pltpu.VMEM((_KE,), jnp.int32),
            pltpu.VMEM((_KE,), jnp.int32),
            pltpu.VMEM((_NP,), jnp.float32),
            pltpu.SemaphoreType.DMA,
            pltpu.SemaphoreType.DMA,
        ],
    )
    def deg_k(ei_hbm, out_hbm, didx0, didx1, degv, semi0, semi1):
        c = lax.axis_index("c")
        s = lax.axis_index("s")
        w = c * _NS + s
        didx = [didx0, didx1]
        semi = [semi0, semi1]
        _fill_1d(degv, _NP, 0.0)
        wbase = w * _NBLK_E
        ones = jnp.ones((_L,), jnp.float32)

        def start(j, blk):
            pltpu.async_copy(ei_hbm.at[wbase + blk, 1], didx[j], semi[j])

        def waiti(j):
            pltpu.make_async_copy(ei_hbm.at[wbase, 1], didx[j], semi[j]).wait()

        def scatter(j):
            for i in range(_KE // _L):
                idx = didx[j][pl.ds(i * _L, _L)]
                plsc.addupdate_scatter(degv, [idx], ones)

        start(0, 0)
        start(1, 1)

        def body(g, carry):
            b0 = 2 * g
            waiti(0)
            scatter(0)
            start(0, b0 + 2)
            waiti(1)
            scatter(1)
            start(1, b0 + 3)
            return carry

        lax.fori_loop(0, _NBLK_E // 2 - 1, body, 0)
        waiti(0)
        scatter(0)
        waiti(1)
        scatter(1)
        pltpu.sync_copy(degv, out_hbm.at[w])

    return deg_k(ei)


def _make_prop(with_scalar):
    out_type = jax.ShapeDtypeStruct((_NC, _RP, _D), jnp.float32)
    if with_scalar:
        out_type = (out_type, jax.ShapeDtypeStruct((_NW, _NP), jnp.float32))
    scratch = [
        pltpu.VMEM((2, _KE), jnp.int32),
        pltpu.VMEM((2, _KE), jnp.int32),
        pltpu.VMEM((_KE, _D), jnp.float32),
        pltpu.VMEM((_KE, _D), jnp.float32),
        pltpu.VMEM_SHARED((_RP, _D), jnp.float32),
        pltpu.SemaphoreType.DMA,
        pltpu.SemaphoreType.DMA,
        pltpu.SemaphoreType.DMA,
        pltpu.SemaphoreType.DMA,
    ]
    if with_scalar:
        scratch += [pltpu.VMEM((_NP,), jnp.float32),
                    pltpu.VMEM((_NP,), jnp.float32)]

    def prop_body(q_hbm, ei_hbm, agg_hbm, s_hbm, idx2_0, idx2_1,
                  rows_0, rows_1, acc, semg0, semg1, sems0, sems1,
                  disv=None, sv=None, dis_hbm=None):
        c = lax.axis_index("c")
        s = lax.axis_index("s")
        w = c * _NS + s
        idx2 = [idx2_0, idx2_1]
        rows = [rows_0, rows_1]
        semg = [semg0, semg1]
        sems = [sems0, sems1]
        _fill_2d(rows_0, _KE, _D, 0.0)
        for j in range(_RT // _KE + 1):
            r0 = j * _KE
            nr = min(_KE, _RT - r0)
            if nr > 0:
                pltpu.sync_copy(rows_0.at[pl.ds(0, nr)],
                                acc.at[pl.ds(s * _RT + r0, nr)])
        if with_scalar:
            _fill_1d(sv, _NP, 0.0)
            pltpu.sync_copy(dis_hbm, disv)
        plsc.subcore_barrier()
        wbase = w * _NBLK_E

        def load_start(j, blk):
            pltpu.sync_copy(ei_hbm.at[wbase + blk], idx2[j])
            pltpu.async_copy(q_hbm.at[idx2[j].at[0]], rows[j], semg[j])

        def waitg(j):
            pltpu.make_async_copy(q_hbm.at[idx2[j].at[0]], rows[j],
                                  semg[j]).wait()

        def starts(j):
            pltpu.async_copy(rows[j], acc.at[idx2[j].at[1]], sems[j],
                             add=True)

        def waits(j):
            pltpu.make_async_copy(rows[j], acc.at[idx2[j].at[1]],
                                  sems[j]).wait()

        def scalars(j):
            for i in range(_KE // _L):
                di = idx2[j][1, pl.ds(i * _L, _L)]
                si = idx2[j][0, pl.ds(i * _L, _L)]
                vals = plsc.load_gather(disv, [di])
                plsc.addupdate_scatter(sv, [si], vals)

        load_start(0, 0)
        load_start(1, 1)

        def body(g, carry):
            b0 = 2 * g
            waitg(0)
            starts(0)
            if with_scalar:
                scalars(0)
            waitg(1)
            starts(1)
            if with_scalar:
                scalars(1)
            waits(0)
            load_start(0, b0 + 2)
            waits(1)
            load_start(1, b0 + 3)
            return carry

        lax.fori_loop(0, _NBLK_E // 2 - 1, body, 0)
        waitg(0)
        starts(0)
        if with_scalar:
            scalars(0)
        waitg(1)
        starts(1)
        if with_scalar:
            scalars(1)
        waits(0)
        waits(1)
        plsc.subcore_barrier()
        pltpu.sync_copy(acc.at[pl.ds(s * _RT, _RT)],
                        agg_hbm.at[c, pl.ds(s * _RT, _RT)])
        if with_scalar:
            pltpu.sync_copy(sv, s_hbm.at[w])

    if with_scalar:
        def prop_k(q_hbm, dis_hbm, ei_hbm, agg_hbm, s_hbm, idx2_0, idx2_1,
                   rows_0, rows_1, acc, semg0, semg1, sems0, sems1,
                   disv, sv):
            prop_body(q_hbm, ei_hbm, agg_hbm, s_hbm, idx2_0, idx2_1,
                      rows_0, rows_1, acc, semg0, semg1, sems0, sems1,
                      disv=disv, sv=sv, dis_hbm=dis_hbm)
    else:
        def prop_k(q_hbm, ei_hbm, agg_hbm, idx2_0, idx2_1,
                   rows_0, rows_1, acc, semg0, semg1, sems0, sems1):
            prop_body(q_hbm, ei_hbm, agg_hbm, None, idx2_0, idx2_1,
                      rows_0, rows_1, acc, semg0, semg1, sems0, sems1)

    return functools.partial(
        pl.kernel,
        out_type=out_type,
        mesh=_mesh(),
        compiler_params=pltpu.CompilerParams(needs_layout_passes=False),
        scratch_types=scratch,
    )(prop_k)


def _prop_scalar_call(q0, dis1, ei):
    return _make_prop(True)(q0, dis1, ei)


def _prop_call(q1, ei):
    return _make_prop(False)(q1, ei)


def _tc1_call(degp, x):
    def body(degp_ref, x_ref, q0_ref, dis16_ref):
        b = pl.program_id(0)
        deg = jnp.sum(degp_ref[...], axis=0)[:, None] + 1.0
        r16 = lax.broadcasted_iota(jnp.int32, (_K, 16), 0) + b * _K
        dis16 = jnp.where(r16 < _N,
                          lax.rsqrt(jnp.broadcast_to(deg, (_K, 16))), 0.0)
        dis16_ref[...] = dis16
        rD = lax.broadcasted_iota(jnp.int32, (_K, _D), 0) + b * _K
        q0_ref[...] = jnp.where(rD < _N, dis16[:, :1] * x_ref[...], 0.0)

    return pl.pallas_call(
        body,
        grid=(_NBLK_N,),
        in_specs=[pl.BlockSpec((_NW, _K), lambda b: (0, b)),
                  pl.BlockSpec((_K, _D), lambda b: (b, 0))],
        out_specs=[pl.BlockSpec((_K, _D), lambda b: (b, 0)),
                   pl.BlockSpec((_K, 16), lambda b: (b, 0))],
        out_shape=[jax.ShapeDtypeStruct((_NP, _D), jnp.float32),
                   jax.ShapeDtypeStruct((_NP, 16), jnp.float32)],
    )(degp, x)


def _tc2_call(aggp, q0, dis16, sp, W0, b0):
    def body(aggp_ref, q0_ref, dis16_ref, sp_ref, W0_ref, b0_ref,
             q1_ref, c16_ref):
        dis = dis16_ref[...]
        dcol = dis[:, :1]
        t = dcol * (aggp_ref[0] + aggp_ref[1] + q0_ref[...])
        h1 = jnp.maximum(
            jnp.dot(t, W0_ref[...], preferred_element_type=jnp.float32)
            + b0_ref[...], 0.0)
        q1_ref[...] = dcol * h1
        sblk = jnp.sum(sp_ref[...], axis=0)[:, None]
        c16_ref[...] = dis * (jnp.broadcast_to(sblk, (_K, 16)) + dis)

    return pl.pallas_call(
        body,
        grid=(_NBLK_N,),
        in_specs=[pl.BlockSpec((_NC, _K, _D), lambda b: (0, b, 0)),
                  pl.BlockSpec((_K, _D), lambda b: (b, 0)),
                  pl.BlockSpec((_K, 16), lambda b: (b, 0)),
                  pl.BlockSpec((_NW, _K), lambda b: (0, b)),
                  pl.BlockSpec((_D, _H), lambda b: (0, 0)),
                  pl.BlockSpec((1, _H), lambda b: (0, 0))],
        out_specs=[pl.BlockSpec((_K, _H), lambda b: (b, 0)),
                   pl.BlockSpec((_K, 16), lambda b: (b, 0))],
        out_shape=[jax.ShapeDtypeStruct((_NP, _H), jnp.float32),
                   jax.ShapeDtypeStruct((_NP, 16), jnp.float32)],
    )(aggp, q0, dis16, sp, W0, b0)


def _tc3_call(aggp, q1, dis16, c16, W1, b1, W2, b2, Wl, bl):
    def body(aggp_ref, q1_ref, dis16_ref, c16_ref, W1_ref, b1_ref,
             W2_ref, b2_ref, Wl_ref, bl_ref, out_ref, u_ref):
        b = pl.program_id(0)
        dcol = dis16_ref[:, :1]
        t = dcol * (aggp_ref[0] + aggp_ref[1] + q1_ref[...])
        h2 = jnp.maximum(
            jnp.dot(t, W1_ref[...], preferred_element_type=jnp.float32)
            + b1_ref[...], 0.0)
        contrib = jnp.sum(c16_ref[:, :1] * h2, axis=0, keepdims=True)

        @pl.when(b == 0)
        def _():
            u_ref[...] = jnp.zeros((1, _H), jnp.float32)

        u_ref[...] = u_ref[...] + contrib

        @pl.when(b == _NBLK_N - 1)
        def _():
            pooled = jnp.dot(u_ref[...], W2_ref[...],
                             preferred_element_type=jnp.float32) \
                + jnp.float32(_N) * b2_ref[...]
            out_ref[...] = jnp.dot(pooled, Wl_ref[...],
                                   preferred_element_type=jnp.float32) \
                + bl_ref[...]

    return pl.pallas_call(
        body,
        grid=(_NBLK_N,),
        in_specs=[pl.BlockSpec((_NC, _K, _D), lambda b: (0, b, 0)),
                  pl.BlockSpec((_K, _D), lambda b: (b, 0)),
                  pl.BlockSpec((_K, 16), lambda b: (b, 0)),
                  pl.BlockSpec((_K, 16), lambda b: (b, 0)),
                  pl.BlockSpec((_H, _H), lambda b: (0, 0)),
                  pl.BlockSpec((1, _H), lambda b: (0, 0)),
                  pl.BlockSpec((_H, _H), lambda b: (0, 0)),
                  pl.BlockSpec((1, _H), lambda b: (0, 0)),
                  pl.BlockSpec((_H, _C), lambda b: (0, 0)),
                  pl.BlockSpec((1, _C), lambda b: (0, 0))],
        out_specs=pl.BlockSpec((1, _C), lambda b: (0, 0)),
        out_shape=jax.ShapeDtypeStruct((1, _C), jnp.float32),
        scratch_shapes=[pltpu.VMEM((1, _H), jnp.float32)],
    )(aggp, q1, dis16, c16, W1, b1, W2, b2, Wl, bl)


def kernel(x, edge_idx, W0, b0, W1, b1, W2, b2, Wl, bl):
    src = edge_idx[0].astype(jnp.int32)
    dst = edge_idx[1].astype(jnp.int32)
    fill = jnp.full((_E_PAD - _E,), _N, jnp.int32)
    src_p = jnp.concatenate([src, fill]).reshape(_NW, _NBLK_E, _KE)
    dst_p = jnp.concatenate([dst, fill]).reshape(_NW, _NBLK_E, _KE)
    ei = jnp.stack([src_p, dst_p], axis=2).reshape(_NW * _NBLK_E, 2, _KE)

    degp = _deg_call(ei)
    q0, dis16 = _tc1_call(degp, x)
    dis1 = dis16[:, 0]
    aggp, sp = _prop_scalar_call(q0, dis1, ei)
    q1, c16 = _tc2_call(aggp, q0, dis16, sp, W0, b0.reshape(1, _H))
    agg1p = _prop_call(q1, ei)
    return _tc3_call(agg1p, q1, dis16, c16, W1, b1.reshape(1, _H),
                     W2, b2.reshape(1, _H), Wl, bl.reshape(1, _C))
